# baseline (speedup 1.0000x reference)
import functools
import math

import jax
import jax.numpy as jnp
from jax import lax
from jax.experimental import pallas as pl
from jax.experimental.pallas import tpu as pltpu
from jax.experimental.pallas import tpu_sc as plsc

D_MODEL = 64
VOCAB = 1000000
SCALE = math.sqrt(D_MODEL)

NC = 2
NS = 16
NW = NC * NS

SEQ = 200
BATCH = 4096
IDXW = 128
CHUNK = 512
GPC = CHUNK // IDXW
QPT = BATCH // CHUNK
BPW = SEQ * QPT // NW
SLAB = CHUNK * D_MODEL // 8


def _emb_kernel(table_hbm, idx_hbm, out_hbm, idx_v, rows_v, stage_v, sem, osem):
    wid = lax.axis_index("s") * NC + lax.axis_index("c")
    lane = lax.broadcasted_iota(jnp.int32, (16,), 0)

    def block_body(bi, _):
        bid = wid * BPW + bi
        t = bid // QPT
        q = lax.rem(bid, QPT)
        pltpu.sync_copy(
            idx_hbm.at[t >> 3, pl.ds(q * GPC, GPC), pl.ds((t & 7) * IDXW, IDXW)],
            idx_v,
        )
        copies = []
        for j in range(GPC):
            copies.append(
                pltpu.async_copy(
                    table_hbm.at[idx_v.at[j]],
                    rows_v.at[pl.ds(j * IDXW, IDXW)],
                    sem,
                )
            )
        for c in copies:
            c.wait()

        def xp_body(sb, _):
            j0 = (sb >> 2) * 16
            d0 = (sb & 3) * 16
            v = [rows_v[j0 + j, pl.ds(d0, 16)] * SCALE for j in range(16)]
            for s in (8, 4, 2, 1):
                perm = lane ^ s
                mask = (lane & s) == 0
                for j in range(16):
                    if j & s:
                        continue
                    p = j | s
                    a, b = v[j], v[p]
                    pa = a.at[perm].get(mode="promise_in_bounds")
                    pb = b.at[perm].get(mode="promise_in_bounds")
                    v[j] = jnp.where(mask, a, pb)
                    v[p] = jnp.where(mask, pa, b)
            rbase = d0 >> 3
            cl = j0 >> 7
            c0 = j0 & 127
            for m in range(16):
                stage_v[rbase + (m >> 3), cl, pl.ds((m & 7) * 128 + c0, 16)] = v[m]
            return 0

        lax.fori_loop(0, 128, xp_body, 0)

        ocopies = []
        for rr in range(8):
            ocopies.append(
                pltpu.async_copy(
                    stage_v.at[rr],
                    out_hbm.at[t, rr, pl.ds(q * GPC, GPC)],
                    osem,
                )
            )
        for c in ocopies:
            c.wait()
        return 0

    lax.fori_loop(0, BPW, block_body, 0)


@jax.jit
def kernel(x, table):
    xt = jnp.swapaxes(x, 0, 1)
    idx3 = xt.reshape(SEQ // 8, 8, BATCH // IDXW, IDXW).transpose(0, 2, 1, 3)
    idx3 = idx3.reshape(SEQ // 8, BATCH // IDXW, 8 * IDXW)
    mesh = plsc.VectorSubcoreMesh(core_axis_name="c", subcore_axis_name="s")
    out3 = pl.kernel(
        _emb_kernel,
        out_type=jax.ShapeDtypeStruct((SEQ, 8, BATCH // IDXW, 8 * IDXW), jnp.float32),
        mesh=mesh,
        scratch_types=[
            pltpu.VMEM((GPC, IDXW), jnp.int32),
            pltpu.VMEM((CHUNK, D_MODEL), jnp.float32),
            pltpu.VMEM((8, GPC, 8 * IDXW), jnp.float32),
            pltpu.SemaphoreType.DMA,
            pltpu.SemaphoreType.DMA,
        ],
        compiler_params=pltpu.CompilerParams(use_tc_tiling_on_sc=False),
    )(table, idx3)
    out5 = out3.reshape(SEQ, 8, BATCH // IDXW, 8, IDXW)
    return lax.reshape(out5, (BATCH, SEQ, D_MODEL), dimensions=(2, 4, 0, 1, 3))


# speedup vs baseline: 1.1232x; 1.1232x over previous
import functools
import math

import jax
import jax.numpy as jnp
from jax import lax
from jax.experimental import pallas as pl
from jax.experimental.pallas import tpu as pltpu
from jax.experimental.pallas import tpu_sc as plsc

D_MODEL = 64
SCALE = math.sqrt(D_MODEL)

NC = 2
NS = 16
NW = NC * NS

SEQ = 200
BATCH = 4096
IDXW = 128
CHUNK = 512
GPC = CHUNK // IDXW
QPT = BATCH // CHUNK
BPW = SEQ * QPT // NW
SLAB = CHUNK * D_MODEL // 8


def _emb_kernel(table_hbm, idx_hbm, out_hbm, idx_v, rows_v, stage_v, sem, osem):
    wid = lax.axis_index("s") * NC + lax.axis_index("c")
    lane = lax.broadcasted_iota(jnp.int32, (16,), 0)

    def block_body(bi, _):
        bid = wid * BPW + bi
        t = bid // QPT
        q = lax.rem(bid, QPT)
        pltpu.sync_copy(
            idx_hbm.at[t >> 3, pl.ds(q * GPC, GPC), pl.ds((t & 7) * IDXW, IDXW)],
            idx_v,
        )
        copies = []
        for j in range(GPC):
            copies.append(
                pltpu.async_copy(
                    table_hbm.at[idx_v.at[j]],
                    rows_v.at[pl.ds(j * IDXW, IDXW)],
                    sem,
                )
            )
        for c in copies:
            c.wait()

        def rot_body(rot, _):
            m = (lane + rot) & 15
            mr = m >> 3
            goff = ((m & 7) << 7) + lane
            for half in range(2):
                for d0g in range(4):
                    d0 = d0g * 16
                    colv = m + d0
                    rv = mr + d0g * 2
                    gathered = []
                    for j0g in range(16):
                        j0 = (half * 16 + j0g) * 16
                        vals = plsc.load_gather(rows_v, [j0 + lane, colv])
                        gathered.append(vals * SCALE)
                    for j0g in range(16):
                        j0 = (half * 16 + j0g) * 16
                        clv = jnp.full((16,), j0 >> 7, jnp.int32)
                        plsc.store_scatter(
                            stage_v, [rv, clv, goff + (j0 & 127)], gathered[j0g]
                        )
            return 0

        lax.fori_loop(0, 16, rot_body, 0)

        ocopies = []
        for rr in range(8):
            ocopies.append(
                pltpu.async_copy(
                    stage_v.at[rr],
                    out_hbm.at[t, rr, pl.ds(q * GPC, GPC)],
                    osem,
                )
            )
        for c in ocopies:
            c.wait()
        return 0

    lax.fori_loop(0, BPW, block_body, 0)


@jax.jit
def kernel(x, table):
    xt = jnp.swapaxes(x, 0, 1)
    idx3 = xt.reshape(SEQ // 8, 8, BATCH // IDXW, IDXW).transpose(0, 2, 1, 3)
    idx3 = idx3.reshape(SEQ // 8, BATCH // IDXW, 8 * IDXW)
    mesh = plsc.VectorSubcoreMesh(core_axis_name="c", subcore_axis_name="s")
    out3 = pl.kernel(
        _emb_kernel,
        out_type=jax.ShapeDtypeStruct((SEQ, 8, BATCH // IDXW, 8 * IDXW), jnp.float32),
        mesh=mesh,
        scratch_types=[
            pltpu.VMEM((GPC, IDXW), jnp.int32),
            pltpu.VMEM((CHUNK, D_MODEL), jnp.float32),
            pltpu.VMEM((8, GPC, 8 * IDXW), jnp.float32),
            pltpu.SemaphoreType.DMA,
            pltpu.SemaphoreType.DMA,
        ],
        compiler_params=pltpu.CompilerParams(
            use_tc_tiling_on_sc=False, needs_layout_passes=False
        ),
    )(table, idx3)
    out5 = out3.reshape(SEQ, 8, BATCH // IDXW, 8, IDXW)
    return lax.reshape(out5, (BATCH, SEQ, D_MODEL), dimensions=(2, 4, 0, 1, 3))
